# baseline (speedup 1.0000x reference)
import functools

import jax
import jax.numpy as jnp
import numpy as np
from jax.experimental import pallas as pl
from jax.experimental.pallas import tpu as pltpu

N = 10000
D_HID = 512
D_OUT = 512
HEADS = 8
LAYERS = 2


def _layer_norm(x, g, b):
    mu = jnp.mean(x, axis=-1, keepdims=True)
    var = jnp.var(x, axis=-1, keepdims=True)
    return (x - mu) / jnp.sqrt(var + 1e-5) * g + b


def _predictor_body(h_ref, w1_ref, b1_ref, w2_ref, b2_ref, out_ref):
    h = h_ref[...]
    t = jnp.maximum(jnp.dot(h, w1_ref[...], preferred_element_type=jnp.float32)
                    + b1_ref[...], 0.0)
    out_ref[...] = jnp.dot(t, w2_ref[...], preferred_element_type=jnp.float32) + b2_ref[...]


def _predictor(h, Wp1, bp1, Wp2, bp2):
    n = h.shape[0]
    blk = 2000
    return pl.pallas_call(
        _predictor_body,
        grid=(n // blk,),
        in_specs=[
            pl.BlockSpec((blk, h.shape[1]), lambda i: (i, 0)),
            pl.BlockSpec(Wp1.shape, lambda i: (0, 0)),
            pl.BlockSpec(bp1.shape, lambda i: (0,)),
            pl.BlockSpec(Wp2.shape, lambda i: (0, 0)),
            pl.BlockSpec(bp2.shape, lambda i: (0,)),
        ],
        out_specs=pl.BlockSpec((blk, Wp2.shape[1]), lambda i: (i, 0)),
        out_shape=jax.ShapeDtypeStruct((n, Wp2.shape[1]), jnp.float32),
    )(h, Wp1, bp1, Wp2, bp2)


def kernel(x, W_gcn1, b_gcn1, W_gcn2, b_gcn2, Wq, Wk, Wv, Wo, ln1_g, ln1_b,
           W_ff1, b_ff1, W_ff2, b_ff2, ln2_g, ln2_b, Wp1, bp1, Wp2, bp2,
           edge_index):
    n = x.shape[0]
    loops = jnp.arange(n, dtype=edge_index.dtype)
    src = jnp.concatenate([edge_index[0], loops])
    dst = jnp.concatenate([edge_index[1], loops])
    deg = jax.ops.segment_sum(jnp.ones_like(src, dtype=jnp.float32), dst, num_segments=n)
    dinv = jax.lax.rsqrt(jnp.maximum(deg, 1.0))
    enorm = dinv[src] * dinv[dst]

    def gcn(h, W, b):
        hw = h @ W
        agg = jax.ops.segment_sum(hw[src] * enorm[:, None], dst, num_segments=n)
        return agg + b

    h = jax.nn.gelu(gcn(x, W_gcn1, b_gcn1))
    h = jax.nn.gelu(gcn(h, W_gcn2, b_gcn2))

    d = D_OUT
    dh = d // HEADS
    for l in range(LAYERS):
        q = (h @ Wq[l]).reshape(n, HEADS, dh)
        k = (h @ Wk[l]).reshape(n, HEADS, dh)
        v = (h @ Wv[l]).reshape(n, HEADS, dh)
        scores = jnp.sum(q[dst] * k[src], axis=-1) / np.sqrt(dh)
        m = jax.ops.segment_max(scores, dst, num_segments=n)
        ex = jnp.exp(scores - m[dst])
        den = jax.ops.segment_sum(ex, dst, num_segments=n)
        alpha = ex / (den[dst] + 1e-9)
        attn = jax.ops.segment_sum(alpha[..., None] * v[src], dst, num_segments=n).reshape(n, d)
        h = _layer_norm(h + attn @ Wo[l], ln1_g[l], ln1_b[l])
        ff = jax.nn.relu(h @ W_ff1[l] + b_ff1[l]) @ W_ff2[l] + b_ff2[l]
        h = _layer_norm(h + ff, ln2_g[l], ln2_b[l])

    x_pred = _predictor(h, Wp1, bp1, Wp2, bp2)
    return (h, x_pred)


# speedup vs baseline: 8.9583x; 8.9583x over previous
import functools

import jax
import jax.numpy as jnp
import numpy as np
from jax import lax
from jax.experimental import pallas as pl
from jax.experimental.pallas import tpu as pltpu
from jax.experimental.pallas import tpu_sc as plsc

N = 10000
E = 160000
D_IN = 256
D = 512
HEADS = 8
DH = 64
LAYERS = 2

NC = 2
NS = 16
NW = NC * NS
EB = E // NW
NB = 40
EBP = NB * 128
ET = NW * EBP
ACC = 10240
FPT = ACC // NS

_MESH = dict(core_axis_name="c", subcore_axis_name="s", num_cores=NC,
             num_subcores=NS)


def _fill_vmem(ref, rows, cols, value):
    nseg = cols // 16

    def body(i, _):
        r = i // nseg
        cc = (i % nseg) * 16
        ref[r, pl.ds(cc, 16)] = jnp.full((16,), value, jnp.float32)
        return 0

    lax.fori_loop(0, rows * nseg, body, 0)


def _deg_body(dstb_hbm, out_hbm, idx_v, val_v, acc_sh):
    c = lax.axis_index("c")
    s = lax.axis_index("s")
    w = c * NS + s
    pltpu.sync_copy(dstb_hbm.at[w], idx_v)
    _fill_vmem(val_v, 128, 16, 0.0)
    for z in range(FPT // 128):
        pltpu.sync_copy(val_v, acc_sh.at[pl.ds(s * FPT + z * 128, 128), :])
    plsc.subcore_barrier()
    _fill_vmem(val_v, 128, 16, 1.0)
    for b in range(NB):
        pltpu.sync_copy(val_v, acc_sh.at[idx_v.at[b]], add=True)
    plsc.subcore_barrier()
    pltpu.sync_copy(acc_sh.at[pl.ds(s * FPT, FPT), :],
                    out_hbm.at[c, pl.ds(s * FPT, FPT), :])


@functools.lru_cache(maxsize=None)
def _deg_kernel():
    return pl.kernel(
        _deg_body,
        out_type=jax.ShapeDtypeStruct((NC, ACC, 16), jnp.float32),
        mesh=plsc.VectorSubcoreMesh(**_MESH),
        scratch_types=[
            pltpu.VMEM((NB, 128), jnp.int32),
            pltpu.VMEM((128, 16), jnp.float32),
            pltpu.VMEM_SHARED((ACC, 16), jnp.float32),
        ],
    )


def _sc_deg(dstb128):
    return _deg_kernel()(dstb128)


def _make_seg_sum(C, gather):
    scratch = [pltpu.VMEM((NB, 128), jnp.int32)]
    if gather:
        scratch.append(pltpu.VMEM((NB, 128), jnp.int32))
    scratch += [
        pltpu.VMEM((128, C), jnp.float32),
        pltpu.VMEM((128, C), jnp.float32),
        pltpu.VMEM_SHARED((ACC, C), jnp.float32),
        pltpu.SemaphoreType.DMA,
        pltpu.SemaphoreType.DMA,
    ]

    def body(*refs):
        if gather:
            (src_hbm, srcb_hbm, dstb_hbm, out_hbm,
             idxd_v, idxs_v, buf0, buf1, acc_sh, sem0, sem1) = refs
        else:
            (src_hbm, dstb_hbm, out_hbm,
             idxd_v, buf0, buf1, acc_sh, sem0, sem1) = refs
        c = lax.axis_index("c")
        s = lax.axis_index("s")
        w = c * NS + s
        pltpu.sync_copy(dstb_hbm.at[w], idxd_v)
        if gather:
            pltpu.sync_copy(srcb_hbm.at[w], idxs_v)
        _fill_vmem(buf0, 128, C, 0.0)
        for z in range(FPT // 128):
            pltpu.sync_copy(buf0, acc_sh.at[pl.ds(s * FPT + z * 128, 128), :])
        plsc.subcore_barrier()

        bufs = (buf0, buf1)
        sems = (sem0, sem1)

        def start(b):
            if gather:
                return pltpu.async_copy(src_hbm.at[idxs_v.at[b]],
                                        bufs[b % 2], sems[b % 2])
            return pltpu.async_copy(
                src_hbm.at[pl.ds(w * EBP + b * 128, 128), :],
                bufs[b % 2], sems[b % 2])

        descs = {0: start(0)}
        for b in range(NB):
            descs.pop(b).wait()
            if b + 1 < NB:
                descs[b + 1] = start(b + 1)
            pltpu.sync_copy(bufs[b % 2], acc_sh.at[idxd_v.at[b]], add=True)
        plsc.subcore_barrier()
        pltpu.sync_copy(acc_sh.at[pl.ds(s * FPT, FPT), :],
                        out_hbm.at[c, pl.ds(s * FPT, FPT), :])

    return pl.kernel(
        body,
        out_type=jax.ShapeDtypeStruct((NC, ACC, C), jnp.float32),
        mesh=plsc.VectorSubcoreMesh(**_MESH),
        scratch_types=scratch,
    )


_make_seg_sum = functools.lru_cache(maxsize=None)(_make_seg_sum)


def _sc_gather_sum(tbl, srcb128, dstb128):
    return _make_seg_sum(128, True)(tbl, srcb128, dstb128)


def _sc_scatter_sum(vals, dstb128):
    return _make_seg_sum(vals.shape[1], False)(vals, dstb128)


_GB = 64
_GNB = EBP // _GB


def _gather3_body(q_hbm, k_hbm, v_hbm, srcb_hbm, dstb_hbm,
                  qd_hbm, ks_hbm, vs_hbm,
                  idxs_v, idxd_v, buf0, buf1, sem0, sem1):
    c = lax.axis_index("c")
    s = lax.axis_index("s")
    w = c * NS + s
    pltpu.sync_copy(srcb_hbm.at[w], idxs_v)
    pltpu.sync_copy(dstb_hbm.at[w], idxd_v)

    def one_pass(tbl, idx, out):
        pltpu.async_copy(tbl.at[idx.at[0]], buf0, sem0)

        def body2(i, _):
            b0 = 2 * i
            b1 = b0 + 1
            pltpu.async_copy(tbl.at[idx.at[b1]], buf1, sem1)
            pltpu.make_async_copy(tbl.at[idx.at[b0]], buf0, sem0).wait()
            pltpu.sync_copy(buf0, out.at[pl.ds(w * EBP + b0 * _GB, _GB), :])

            @pl.when(i < _GNB // 2 - 1)
            def _():
                pltpu.async_copy(tbl.at[idx.at[b0 + 2]], buf0, sem0)

            pltpu.make_async_copy(tbl.at[idx.at[b1]], buf1, sem1).wait()
            pltpu.sync_copy(buf1, out.at[pl.ds(w * EBP + b1 * _GB, _GB), :])
            return 0

        lax.fori_loop(0, _GNB // 2, body2, 0)

    one_pass(q_hbm, idxd_v, qd_hbm)
    one_pass(k_hbm, idxs_v, ks_hbm)
    one_pass(v_hbm, idxs_v, vs_hbm)


@functools.lru_cache(maxsize=None)
def _gather3_kernel():
    return pl.kernel(
        _gather3_body,
        out_type=(jax.ShapeDtypeStruct((ET, D), jnp.float32),
                  jax.ShapeDtypeStruct((ET, D), jnp.float32),
                  jax.ShapeDtypeStruct((ET, D), jnp.float32)),
        mesh=plsc.VectorSubcoreMesh(**_MESH),
        scratch_types=[
            pltpu.VMEM((_GNB, _GB), jnp.int32),
            pltpu.VMEM((_GNB, _GB), jnp.int32),
            pltpu.VMEM((_GB, D), jnp.float32),
            pltpu.VMEM((_GB, D), jnp.float32),
            pltpu.SemaphoreType.DMA,
            pltpu.SemaphoreType.DMA,
        ],
    )


def _sc_gather3(q, k, v, srcb64, dstb64):
    return _gather3_kernel()(q, k, v, srcb64, dstb64)


BLK = 2000
BLK7 = 1000
EBLK = 2048


def _ln(x, g, b):
    mu = jnp.mean(x, axis=-1, keepdims=True)
    xc = x - mu
    var = jnp.mean(xc * xc, axis=-1, keepdims=True)
    return xc * lax.rsqrt(var + 1e-5) * g + b


def _dot(a, b):
    return jnp.dot(a, b, preferred_element_type=jnp.float32)


def _t1_body(x_ref, w_ref, degp_ref, c0, c1, c2, c3, dinv_ref):
    xw = _dot(x_ref[...], w_ref[...])
    deg = degp_ref[0, :, 0:1] + degp_ref[1, :, 0:1] + 1.0
    dinv = lax.rsqrt(deg)
    hwp = xw * dinv
    outs = (c0, c1, c2, c3)
    for j in range(4):
        outs[j][...] = hwp[:, j * 128:(j + 1) * 128]
    dinv_ref[...] = dinv


def _t1(x, W1, degp):
    return pl.pallas_call(
        _t1_body,
        grid=(N // BLK,),
        in_specs=[
            pl.BlockSpec((BLK, D_IN), lambda i: (i, 0)),
            pl.BlockSpec((D_IN, D), lambda i: (0, 0)),
            pl.BlockSpec((2, BLK, 16), lambda i: (0, i, 0)),
        ],
        out_specs=[pl.BlockSpec((BLK, 128), lambda i: (i, 0))] * 4
        + [pl.BlockSpec((BLK, 1), lambda i: (i, 0))],
        out_shape=[jax.ShapeDtypeStruct((N, 128), jnp.float32)] * 4
        + [jax.ShapeDtypeStruct((N, 1), jnp.float32)],
    )(x, W1, degp)


def _t3_body(h_ref, w_ref, dinv_ref, c0, c1, c2, c3):
    hw = _dot(h_ref[...], w_ref[...])
    hwp = hw * dinv_ref[...]
    outs = (c0, c1, c2, c3)
    for j in range(4):
        outs[j][...] = hwp[:, j * 128:(j + 1) * 128]


def _t3(h, W2, dinv):
    return pl.pallas_call(
        _t3_body,
        grid=(N // BLK,),
        in_specs=[
            pl.BlockSpec((BLK, D), lambda i: (i, 0)),
            pl.BlockSpec((D, D), lambda i: (0, 0)),
            pl.BlockSpec((BLK, 1), lambda i: (i, 0)),
        ],
        out_specs=[pl.BlockSpec((BLK, 128), lambda i: (i, 0))] * 4,
        out_shape=[jax.ShapeDtypeStruct((N, 128), jnp.float32)] * 4,
    )(h, W2, dinv)


def _t2_body(a0, a1, a2, a3, c0, c1, c2, c3, dinv_ref, b_ref, out_ref):
    aggs = [r[0] + r[1] for r in (a0, a1, a2, a3)]
    hwps = [r[...] for r in (c0, c1, c2, c3)]
    agg = jnp.concatenate(aggs, axis=1)
    hwp = jnp.concatenate(hwps, axis=1)
    out_ref[...] = jax.nn.gelu(dinv_ref[...] * (agg + hwp) + b_ref[...])


def _t2(agg_chunks, hwp_chunks, dinv, bias):
    return pl.pallas_call(
        _t2_body,
        grid=(N // BLK,),
        in_specs=[pl.BlockSpec((2, BLK, 128), lambda i: (0, i, 0))] * 4
        + [pl.BlockSpec((BLK, 128), lambda i: (i, 0))] * 4
        + [
            pl.BlockSpec((BLK, 1), lambda i: (i, 0)),
            pl.BlockSpec((1, D), lambda i: (0, 0)),
        ],
        out_specs=pl.BlockSpec((BLK, D), lambda i: (i, 0)),
        out_shape=jax.ShapeDtypeStruct((N, D), jnp.float32),
    )(*agg_chunks, *hwp_chunks, dinv, bias)


def _t5_body(h_ref, wq_ref, wk_ref, wv_ref, bd_ref, q_ref, k_ref, v_ref,
             es_ref):
    h = h_ref[...]
    q = _dot(h, wq_ref[...])
    k = _dot(h, wk_ref[...])
    v = _dot(h, wv_ref[...])
    q_ref[...] = q
    k_ref[...] = k
    v_ref[...] = v
    es_ref[...] = jnp.exp(_dot(q * k, bd_ref[...]) * 0.125)


def _t5(h, Wq, Wk, Wv, bd):
    return pl.pallas_call(
        _t5_body,
        grid=(N // BLK,),
        in_specs=[
            pl.BlockSpec((BLK, D), lambda i: (i, 0)),
            pl.BlockSpec((D, D), lambda i: (0, 0)),
            pl.BlockSpec((D, D), lambda i: (0, 0)),
            pl.BlockSpec((D, D), lambda i: (0, 0)),
            pl.BlockSpec((D, 8), lambda i: (0, 0)),
        ],
        out_specs=[pl.BlockSpec((BLK, D), lambda i: (i, 0))] * 3
        + [pl.BlockSpec((BLK, 8), lambda i: (i, 0))],
        out_shape=[jax.ShapeDtypeStruct((N, D), jnp.float32)] * 3
        + [jax.ShapeDtypeStruct((N, 8), jnp.float32)],
    )(h, Wq, Wk, Wv, bd)


def _t6_body(qd_ref, ks_ref, vs_ref, bd_ref, ex_ref, w0, w1, w2, w3):
    prod = qd_ref[...] * ks_ref[...]
    ex = jnp.exp(_dot(prod, bd_ref[...]) * 0.125)
    ex_ref[...] = jnp.concatenate([ex, ex], axis=1)
    vs = vs_ref[...]
    outs = (w0, w1, w2, w3)
    for j in range(4):
        b = j * 128
        outs[j][...] = jnp.concatenate(
            [vs[:, b:b + 64] * ex[:, 2 * j:2 * j + 1],
             vs[:, b + 64:b + 128] * ex[:, 2 * j + 1:2 * j + 2]], axis=1)


def _t6(qd, ks, vs, bd):
    return pl.pallas_call(
        _t6_body,
        grid=(ET // EBLK,),
        in_specs=[
            pl.BlockSpec((EBLK, D), lambda i: (i, 0)),
            pl.BlockSpec((EBLK, D), lambda i: (i, 0)),
            pl.BlockSpec((EBLK, D), lambda i: (i, 0)),
            pl.BlockSpec((D, 8), lambda i: (0, 0)),
        ],
        out_specs=[pl.BlockSpec((EBLK, 16), lambda i: (i, 0))]
        + [pl.BlockSpec((EBLK, 128), lambda i: (i, 0))] * 4,
        out_shape=[jax.ShapeDtypeStruct((ET, 16), jnp.float32)]
        + [jax.ShapeDtypeStruct((ET, 128), jnp.float32)] * 4,
    )(qd, ks, vs, bd)


def _t7_body(h_ref, n0, n1, n2, n3, denp_ref, es_ref, v_ref, wo_ref,
             g1_ref, b1_ref, wf1_ref, bf1_ref, wf2_ref, bf2_ref,
             g2_ref, b2_ref, out_ref):
    h = h_ref[...]
    num = jnp.concatenate([r[0] + r[1] for r in (n0, n1, n2, n3)], axis=1)
    den = denp_ref[0, :, 0:8] + denp_ref[1, :, 0:8]
    es = es_ref[...]
    v = v_ref[...]
    dentot = den + es + 1e-30
    segs = []
    for hh in range(HEADS):
        b = hh * DH
        numh = num[:, b:b + DH] + es[:, hh:hh + 1] * v[:, b:b + DH]
        segs.append(numh / dentot[:, hh:hh + 1])
    attn = jnp.concatenate(segs, axis=1)
    o = h + _dot(attn, wo_ref[...])
    u = _ln(o, g1_ref[...], b1_ref[...])
    ff = _dot(jnp.maximum(_dot(u, wf1_ref[...]) + bf1_ref[...], 0.0),
              wf2_ref[...]) + bf2_ref[...]
    out_ref[...] = _ln(u + ff, g2_ref[...], b2_ref[...])


def _t7(h, num_chunks, denp, es, v, Wo, g1, b1, Wf1, bf1, Wf2, bf2, g2, b2):
    return pl.pallas_call(
        _t7_body,
        grid=(N // BLK7,),
        in_specs=[pl.BlockSpec((BLK7, D), lambda i: (i, 0))]
        + [pl.BlockSpec((2, BLK7, 128), lambda i: (0, i, 0))] * 4
        + [
            pl.BlockSpec((2, BLK7, 16), lambda i: (0, i, 0)),
            pl.BlockSpec((BLK7, 8), lambda i: (i, 0)),
            pl.BlockSpec((BLK7, D), lambda i: (i, 0)),
            pl.BlockSpec((D, D), lambda i: (0, 0)),
            pl.BlockSpec((1, D), lambda i: (0, 0)),
            pl.BlockSpec((1, D), lambda i: (0, 0)),
            pl.BlockSpec((D, 4 * D), lambda i: (0, 0)),
            pl.BlockSpec((1, 4 * D), lambda i: (0, 0)),
            pl.BlockSpec((4 * D, D), lambda i: (0, 0)),
            pl.BlockSpec((1, D), lambda i: (0, 0)),
            pl.BlockSpec((1, D), lambda i: (0, 0)),
            pl.BlockSpec((1, D), lambda i: (0, 0)),
        ],
        out_specs=pl.BlockSpec((BLK7, D), lambda i: (i, 0)),
        out_shape=jax.ShapeDtypeStruct((N, D), jnp.float32),
    )(h, *num_chunks, denp, es, v, Wo, g1, b1, Wf1, bf1, Wf2, bf2, g2, b2)


def _t8_body(h_ref, w1_ref, b1_ref, w2_ref, b2_ref, out_ref):
    t = jnp.maximum(_dot(h_ref[...], w1_ref[...]) + b1_ref[...], 0.0)
    out_ref[...] = _dot(t, w2_ref[...]) + b2_ref[...]


def _t8(h, Wp1, bp1, Wp2, bp2):
    return pl.pallas_call(
        _t8_body,
        grid=(N // BLK,),
        in_specs=[
            pl.BlockSpec((BLK, D), lambda i: (i, 0)),
            pl.BlockSpec((D, D // 2), lambda i: (0, 0)),
            pl.BlockSpec((1, D // 2), lambda i: (0, 0)),
            pl.BlockSpec((D // 2, D), lambda i: (0, 0)),
            pl.BlockSpec((1, D), lambda i: (0, 0)),
        ],
        out_specs=pl.BlockSpec((BLK, D), lambda i: (i, 0)),
        out_shape=jax.ShapeDtypeStruct((N, D), jnp.float32),
    )(h, Wp1, bp1, Wp2, bp2)


def kernel(x, W_gcn1, b_gcn1, W_gcn2, b_gcn2, Wq, Wk, Wv, Wo, ln1_g, ln1_b,
           W_ff1, b_ff1, W_ff2, b_ff2, ln2_g, ln2_b, Wp1, bp1, Wp2, bp2,
           edge_index):
    src = edge_index[0]
    dst = edge_index[1]
    srcb = jnp.pad(src.reshape(NW, EB), ((0, 0), (0, EBP - EB)),
                   constant_values=0)
    dstb = jnp.pad(dst.reshape(NW, EB), ((0, 0), (0, EBP - EB)),
                   constant_values=N)
    srcb128 = srcb.reshape(NW, NB, 128)
    dstb128 = dstb.reshape(NW, NB, 128)
    srcb64 = srcb.reshape(NW, _GNB, _GB)
    dstb64 = dstb.reshape(NW, _GNB, _GB)

    bd = jnp.asarray(
        (np.arange(D)[:, None] // DH == np.arange(HEADS)[None, :])
        .astype(np.float32))

    def r2(a):
        return a.reshape(1, -1)

    degp = _sc_deg(dstb128)[:, :N, :]

    *hwp1, dinv = _t1(x, W_gcn1, degp)
    agg1 = [_sc_gather_sum(hwp1[j], srcb128, dstb128)[:, :N, :]
            for j in range(4)]
    h = _t2(agg1, hwp1, dinv, r2(b_gcn1))

    hwp2 = _t3(h, W_gcn2, dinv)
    agg2 = [_sc_gather_sum(hwp2[j], srcb128, dstb128)[:, :N, :]
            for j in range(4)]
    h = _t2(agg2, hwp2, dinv, r2(b_gcn2))

    for l in range(LAYERS):
        q, k, v, es = _t5(h, Wq[l], Wk[l], Wv[l], bd)
        qd, ks, vs = _sc_gather3(q, k, v, srcb64, dstb64)
        ex16, w0, w1, w2, w3 = _t6(qd, ks, vs, bd)
        denp = _sc_scatter_sum(ex16, dstb128)[:, :N, :]
        nump = [_sc_scatter_sum(wj, dstb128)[:, :N, :]
                for wj in (w0, w1, w2, w3)]
        h = _t7(h, nump, denp, es, v, Wo[l], r2(ln1_g[l]), r2(ln1_b[l]),
                W_ff1[l], r2(b_ff1[l]), W_ff2[l], r2(b_ff2[l]),
                r2(ln2_g[l]), r2(ln2_b[l]))

    x_pred = _t8(h, Wp1, r2(bp1), Wp2, r2(bp2))
    return (h, x_pred)
